# baseline (speedup 1.0000x reference)
import functools

import jax
import jax.numpy as jnp
from jax import lax
from jax.experimental import pallas as pl
from jax.experimental.pallas import tpu as pltpu
from jax.experimental.pallas import tpu_sc as plsc

BATCH = 4
SEQLEN = 2048
EMBED = 64
NUM_CORES = 2
NUM_SUBCORES = 16
NW = NUM_CORES * NUM_SUBCORES
TOTAL = BATCH * SEQLEN
CHUNK = TOTAL // NW
WAVE = 16
NWAVE = CHUNK // WAVE
LANES = 16
BLANES = 32
SUBS = 16


@functools.partial(
    pl.kernel,
    out_type=jax.ShapeDtypeStruct((TOTAL, EMBED), jnp.bfloat16),
    mesh=plsc.VectorSubcoreMesh(core_axis_name="c", subcore_axis_name="s"),
    scratch_types=[
        pltpu.VMEM((CHUNK + LANES,), jnp.int32),
        pltpu.VMEM((2, WAVE, SUBS, EMBED), jnp.bfloat16),
        pltpu.VMEM((CHUNK, EMBED), jnp.bfloat16),
        pltpu.VMEM((CHUNK, EMBED), jnp.bfloat16),
        pltpu.SemaphoreType.DMA,
        pltpu.SemaphoreType.DMA,
        pltpu.SemaphoreType.DMA,
    ],
    compiler_params=pltpu.CompilerParams(needs_layout_passes=False),
)
def _embed_lookup(x_hbm, tab_hbm, pos_hbm, out_hbm,
                  idx_v, prow_v, pos_v, out_v, sem_a, sem_b, psem):
    wid = lax.axis_index("s") * NUM_CORES + lax.axis_index("c")
    base = wid * CHUNK
    pos_base = lax.rem(base, SEQLEN)

    pltpu.sync_copy(x_hbm.at[pl.ds(base, CHUNK)], idx_v.at[pl.ds(0, CHUNK)])
    pos_cp = pltpu.async_copy(pos_hbm.at[pl.ds(pos_base, CHUNK)], pos_v, psem)
    sems = (sem_a, sem_b)

    def fire_wave(w):
        b = w % 2

        def fire(j, carry):
            tok = idx_v[pl.ds(w * WAVE + j, LANES)][0]
            g = lax.shift_right_logical(tok, 4)
            pltpu.async_copy(tab_hbm.at[g], prow_v.at[b, j], sems[b])
            return carry

        lax.fori_loop(0, WAVE, fire, 0)

    def drain_wave(w):
        b = w % 2

        def drain(j, carry):
            pltpu.make_async_copy(
                tab_hbm.at[0], prow_v.at[b, j], sems[b]
            ).wait()
            return carry

        lax.fori_loop(0, WAVE, drain, 0)

    def compute_wave(w):
        b = w % 2

        def comp(j, carry):
            t = w * WAVE + j
            tok = idx_v[pl.ds(t, LANES)][0]
            s = tok & (SUBS - 1)
            for c in range(EMBED // BLANES):
                sl = pl.ds(c * BLANES, BLANES)
                out_v[t, sl] = prow_v[b, j, s, sl] + pos_v[t, sl]
            return carry

        lax.fori_loop(0, WAVE, comp, 0)

    fire_wave(0)
    pos_cp.wait()
    for w in range(NWAVE):
        if w + 1 < NWAVE:
            fire_wave(w + 1)
        drain_wave(w)
        compute_wave(w)

    pltpu.sync_copy(out_v, out_hbm.at[pl.ds(base, CHUNK)])


VOCAB = 1000000
TXW = 4096


def _tx_body(inT_ref, out_ref):
    out_ref[...] = inT_ref[...].T.astype(jnp.bfloat16)


_transpose_cast = pl.pallas_call(
    _tx_body,
    grid=(pl.cdiv(VOCAB, TXW),),
    in_specs=[pl.BlockSpec((EMBED, TXW), lambda g: (0, g))],
    out_specs=pl.BlockSpec((TXW, EMBED), lambda g: (g, 0)),
    out_shape=jax.ShapeDtypeStruct((VOCAB, EMBED), jnp.bfloat16),
)


def kernel(x, token_table, pos_table):
    tok16 = _transpose_cast(token_table.T).reshape(VOCAB // SUBS, SUBS, EMBED)
    pos16 = pos_table.astype(jnp.bfloat16)
    out = _embed_lookup(x.reshape(TOTAL), tok16, pos16)
    return out.reshape(BATCH, SEQLEN, EMBED)


# speedup vs baseline: 1.0302x; 1.0302x over previous
import functools

import jax
import jax.numpy as jnp
from jax import lax
from jax.experimental import pallas as pl
from jax.experimental.pallas import tpu as pltpu
from jax.experimental.pallas import tpu_sc as plsc

BATCH = 4
SEQLEN = 2048
EMBED = 64
NUM_CORES = 2
NUM_SUBCORES = 16
NW = NUM_CORES * NUM_SUBCORES
TOTAL = BATCH * SEQLEN
CHUNK = TOTAL // NW
WAVE = 16
NWAVE = CHUNK // WAVE
LANES = 16
BLANES = 32
SUBS = 16
VOCAB = 1000000
TXW = 2048
NTX = 245
TROWS = NTX * TXW


@functools.partial(
    pl.kernel,
    out_type=jax.ShapeDtypeStruct((TOTAL, EMBED), jnp.bfloat16),
    mesh=plsc.VectorSubcoreMesh(core_axis_name="c", subcore_axis_name="s"),
    scratch_types=[
        pltpu.VMEM((CHUNK + LANES,), jnp.int32),
        pltpu.VMEM((2, WAVE, SUBS, 2 * EMBED), jnp.bfloat16),
        pltpu.VMEM((CHUNK, EMBED), jnp.bfloat16),
        pltpu.VMEM((CHUNK, EMBED), jnp.bfloat16),
        pltpu.SemaphoreType.DMA,
        pltpu.SemaphoreType.DMA,
        pltpu.SemaphoreType.DMA,
    ],
    compiler_params=pltpu.CompilerParams(needs_layout_passes=False),
)
def _embed_lookup(x_hbm, tab_hbm, pos_hbm, out_hbm,
                  idx_v, prow_v, pos_v, out_v, sem_a, sem_b, psem):
    wid = lax.axis_index("s") * NUM_CORES + lax.axis_index("c")
    base = wid * CHUNK
    pos_base = lax.rem(base, SEQLEN)

    pltpu.sync_copy(x_hbm.at[pl.ds(base, CHUNK)], idx_v.at[pl.ds(0, CHUNK)])
    pos_cp = pltpu.async_copy(pos_hbm.at[pl.ds(pos_base, CHUNK)], pos_v, psem)
    sems = (sem_a, sem_b)

    def fire_wave(w):
        b = w % 2

        def fire(j, carry):
            tok = idx_v[pl.ds(w * WAVE + j, LANES)][0]
            row = (lax.shift_right_logical(tok, 12) * 2048) + (tok & 2047)
            g = lax.shift_right_logical(row, 4)
            pltpu.async_copy(tab_hbm.at[g], prow_v.at[b, j], sems[b])
            return carry

        lax.fori_loop(0, WAVE, fire, 0)

    def drain_wave(w):
        b = w % 2

        def drain(j, carry):
            pltpu.make_async_copy(
                tab_hbm.at[0], prow_v.at[b, j], sems[b]
            ).wait()
            return carry

        lax.fori_loop(0, WAVE, drain, 0)

    def compute_wave(w):
        b = w % 2

        def comp(j, carry):
            t = w * WAVE + j
            tok = idx_v[pl.ds(t, LANES)][0]
            s = tok & (SUBS - 1)
            h = (lax.shift_right_logical(tok, 11) & 1) * EMBED
            for c in range(EMBED // BLANES):
                out_v[t, pl.ds(c * BLANES, BLANES)] = (
                    prow_v[b, j, s, pl.ds(h + c * BLANES, BLANES)]
                    + pos_v[t, pl.ds(c * BLANES, BLANES)]
                )
            return carry

        lax.fori_loop(0, WAVE, comp, 0)

    fire_wave(0)
    pos_cp.wait()
    for w in range(NWAVE):
        if w + 1 < NWAVE:
            fire_wave(w + 1)
        drain_wave(w)
        compute_wave(w)

    pltpu.sync_copy(out_v, out_hbm.at[pl.ds(base, CHUNK)])


def _tx_body(inA_ref, inB_ref, out_ref):
    r = lax.broadcasted_iota(jnp.int32, (EMBED, EMBED), 0)
    c = lax.broadcasted_iota(jnp.int32, (EMBED, EMBED), 1)
    ident = (r == c).astype(jnp.bfloat16)
    dn = (((0,), (0,)), ((), ()))
    ya = lax.dot_general(
        inA_ref[...].astype(jnp.bfloat16), ident, dn,
        preferred_element_type=jnp.float32,
    )
    yb = lax.dot_general(
        inB_ref[...].astype(jnp.bfloat16), ident, dn,
        preferred_element_type=jnp.float32,
    )
    out_ref[...] = jnp.concatenate(
        [ya.astype(jnp.bfloat16), yb.astype(jnp.bfloat16)], axis=1
    )


_transpose_cast = pl.pallas_call(
    _tx_body,
    grid=(NTX,),
    in_specs=[
        pl.BlockSpec((EMBED, TXW), lambda g: (0, 2 * g)),
        pl.BlockSpec((EMBED, TXW), lambda g: (0, jnp.minimum(2 * g + 1, VOCAB // TXW))),
    ],
    out_specs=pl.BlockSpec((TXW, 2 * EMBED), lambda g: (g, 0)),
    out_shape=jax.ShapeDtypeStruct((TROWS, 2 * EMBED), jnp.bfloat16),
)


def kernel(x, token_table, pos_table):
    tokT = token_table.T
    tok16 = _transpose_cast(tokT, tokT).reshape(TROWS // SUBS, SUBS, 2 * EMBED)
    pos16 = pos_table.astype(jnp.bfloat16)
    out = _embed_lookup(x.reshape(TOTAL), tok16, pos16)
    return out.reshape(BATCH, SEQLEN, EMBED)


# speedup vs baseline: 2.0451x; 1.9852x over previous
import functools

import jax
import jax.numpy as jnp
from jax import lax
from jax.experimental import pallas as pl
from jax.experimental.pallas import tpu as pltpu
from jax.experimental.pallas import tpu_sc as plsc

BATCH = 4
SEQLEN = 2048
EMBED = 64
NUM_CORES = 2
NUM_SUBCORES = 16
NW = NUM_CORES * NUM_SUBCORES
TOTAL = BATCH * SEQLEN
CHUNK = TOTAL // NW
WAVE = 4
NWAVE = CHUNK // WAVE
LANES = 16
TCOL = 128
VOCAB = 1000000
TAIL_START = (VOCAB // TCOL) * TCOL


@functools.partial(
    pl.kernel,
    out_type=jax.ShapeDtypeStruct((EMBED, TOTAL), jnp.float32),
    mesh=plsc.VectorSubcoreMesh(core_axis_name="c", subcore_axis_name="s"),
    scratch_types=[
        pltpu.VMEM((CHUNK + LANES,), jnp.int32),
        pltpu.VMEM((2, WAVE, EMBED, TCOL), jnp.float32),
        pltpu.VMEM((EMBED, CHUNK), jnp.float32),
        pltpu.VMEM((EMBED, CHUNK), jnp.float32),
        pltpu.SemaphoreType.DMA,
        pltpu.SemaphoreType.DMA,
        pltpu.SemaphoreType.DMA,
    ],
    compiler_params=pltpu.CompilerParams(needs_layout_passes=False),
)
def _embed_lookup(x_hbm, tokT_hbm, tailT_hbm, posT_hbm, outT_hbm,
                  idx_v, win_v, posT_v, outT_v, sem_a, sem_b, psem):
    wid = lax.axis_index("s") * NUM_CORES + lax.axis_index("c")
    base = wid * CHUNK
    pos_base = lax.rem(base, SEQLEN)

    pltpu.sync_copy(x_hbm.at[pl.ds(base, CHUNK)], idx_v.at[pl.ds(0, CHUNK)])
    pos_cp = pltpu.async_copy(
        posT_hbm.at[:, pl.ds(pos_base, CHUNK)], posT_v, psem
    )
    sems = (sem_a, sem_b)
    iota = lax.iota(jnp.int32, LANES)

    def fire_wave(w):
        b = w % 2

        def fire(j, carry):
            tok = idx_v[pl.ds(w * WAVE + j, LANES)][0]
            g = lax.shift_right_logical(tok, 7)

            @pl.when(tok < TAIL_START)
            def _():
                pltpu.async_copy(
                    tokT_hbm.at[:, pl.ds(g * TCOL, TCOL)],
                    win_v.at[b, j], sems[b],
                )

            @pl.when(tok >= TAIL_START)
            def _():
                pltpu.async_copy(tailT_hbm.at[:, :], win_v.at[b, j], sems[b])

            return carry

        lax.fori_loop(0, WAVE, fire, 0)

    def drain_wave(w):
        b = w % 2

        def drain(j, carry):
            pltpu.make_async_copy(
                tailT_hbm.at[:, :], win_v.at[b, j], sems[b]
            ).wait()
            return carry

        lax.fori_loop(0, WAVE, drain, 0)

    def compute_wave(w):
        b = w % 2

        def comp(j, carry):
            t = w * WAVE + j
            tok = idx_v[pl.ds(t, LANES)][0]
            col = iota * 0 + (tok & (TCOL - 1))
            slot = iota * 0 + j
            tv = iota * 0 + t
            for k in range(EMBED // LANES):
                rows = iota + k * LANES
                vals = plsc.load_gather(win_v, [slot * 0 + b, slot, rows, col])
                pv = plsc.load_gather(posT_v, [rows, tv])
                plsc.store_scatter(outT_v, [rows, tv], vals + pv)
            return carry

        lax.fori_loop(0, WAVE, comp, 0)

    fire_wave(0)
    pos_cp.wait()
    for w in range(NWAVE):
        if w + 1 < NWAVE:
            fire_wave(w + 1)
        drain_wave(w)
        compute_wave(w)

    pltpu.sync_copy(outT_v, outT_hbm.at[:, pl.ds(base, CHUNK)])


def kernel(x, token_table, pos_table):
    tokT = token_table.T
    tailT = jnp.pad(token_table[TAIL_START:].T, ((0, 0), (0, TCOL - (VOCAB - TAIL_START))))
    outT = _embed_lookup(x.reshape(TOTAL), tokT, tailT, pos_table.T)
    return outT.T.reshape(BATCH, SEQLEN, EMBED).astype(jnp.bfloat16)
